# Initial kernel scaffold; baseline (speedup 1.0000x reference)
import jax
import jax.numpy as jnp
from jax.experimental import pallas as pl


def kernel(entity_emb, relation_emb, W, edge_index, edge_type):
    raise NotImplementedError("write your pallas kernel here")


# speedup vs baseline: 6.7986x; 6.7986x over previous
import functools

import jax
import jax.numpy as jnp
from jax import lax
from jax.experimental import pallas as pl
from jax.experimental.pallas import tpu as pltpu
from jax.experimental.pallas import tpu_sc as plsc

NC = 2
NS = 16
NW = NC * NS
L = 16
CH = 128
NEG = -3.38e38


def _iota():
  return lax.iota(jnp.int32, L)


def _seg_rmw(h16, v16, acc_ref, kscr, vscr, is_max):
  h_s, v_s = plsc.sort_key_val(h16, v16)
  ii = _iota()
  kscr[...] = h_s
  h_prev = plsc.load_gather(kscr, [jnp.maximum(ii - 1, 0)])
  is_start = (ii == 0) | (h_s != h_prev)
  rs = plsc.cummax(jnp.where(is_start, ii, -1))
  m = v_s
  for k in (1, 2, 4, 8):
    vscr[...] = m
    prev = plsc.load_gather(vscr, [jnp.maximum(ii - k, 0)])
    valid = (ii - k) >= rs
    if is_max:
      m = jnp.where(valid, jnp.maximum(m, prev), m)
    else:
      m = jnp.where(valid, m + prev, m)
  h_next = plsc.load_gather(kscr, [jnp.minimum(ii + 1, L - 1)])
  is_last = (ii == L - 1) | (h_s != h_next)
  cur = plsc.load_gather(acc_ref, [h_s])
  upd = jnp.maximum(cur, m) if is_max else cur + m
  plsc.store_scatter(acc_ref, [h_s], upd, mask=is_last)


def _chunk_gather(src_hbm, idx_ref, dst_ref, sem, nch):
  G = 8

  def gbody(g, _):
    descs = []
    for jj in range(G):
      off = (g * G + jj) * CH
      descs.append(
          pltpu.async_copy(
              src_hbm.at[idx_ref.at[pl.ds(off, CH)]],
              dst_ref.at[pl.ds(off, CH)], sem))
    for d in descs:
      d.wait()
    return 0

  lax.fori_loop(0, nch // G, gbody, 0, unroll=False)


def _wid():
  return lax.axis_index("s") * NC + lax.axis_index("c")


def kernel(entity_emb, relation_emb, W, edge_index, edge_type):
  N, D = entity_emb.shape
  R = relation_emb.shape[0]
  E = edge_index.shape[1]
  n_hops = 2
  res_lambda = 0.5

  NCH = -(-E // (NW * CH))
  NCH = -(-NCH // 8) * 8
  EWP = NCH * CH
  EP = NW * EWP
  NP = -(-(N + 1) // (NW * CH)) * (NW * CH)
  SL = NP // NW
  NPO = -(-(N + 1) // CH) * CH
  SLC = NPO // NS
  NV = EWP // L

  mesh = plsc.VectorSubcoreMesh(
      core_axis_name="c", subcore_axis_name="s",
      num_cores=NC, num_subcores=NS)
  f32 = jnp.float32
  i32 = jnp.int32

  @functools.partial(
      pl.kernel,
      out_type=(jax.ShapeDtypeStruct((EP,), f32),
                jax.ShapeDtypeStruct((NW, NP), f32)),
      mesh=mesh,
      compiler_params=pltpu.CompilerParams(needs_layout_passes=False),
      scratch_types=[
          pltpu.VMEM((EWP,), i32),
          pltpu.VMEM((EWP,), f32),
          pltpu.VMEM((EWP,), f32),
          pltpu.VMEM((EWP,), i32),
          pltpu.VMEM((EWP,), f32),
          pltpu.VMEM((NP,), f32),
          pltpu.VMEM((L,), i32),
          pltpu.VMEM((L,), f32),
          pltpu.SemaphoreType.DMA,
      ])
  def ka(idx1_h, idx2_h, h_h, ab_h, e_o, m_o,
         idxb, va, vb, hv, ev, mloc, kscr, vscr, sem):
    w = _wid()
    base = w * EWP
    pltpu.sync_copy(idx1_h.at[pl.ds(base, EWP)], idxb)
    _chunk_gather(ab_h, idxb, va, sem, NCH)
    pltpu.sync_copy(idx2_h.at[pl.ds(base, EWP)], idxb)
    _chunk_gather(ab_h, idxb, vb, sem, NCH)
    pltpu.sync_copy(h_h.at[pl.ds(base, EWP)], hv)

    def zbody(i, _):
      mloc[pl.ds(i * L, L)] = jnp.full((L,), NEG, f32)
      return 0
    lax.fori_loop(0, NP // L, zbody, 0, unroll=False)

    def vbody(i, _):
      off = i * L
      x = va[pl.ds(off, L)] + vb[pl.ds(off, L)]
      e16 = jnp.where(x > 0, x, 0.2 * x)
      ev[pl.ds(off, L)] = e16
      _seg_rmw(hv[pl.ds(off, L)], e16, mloc, kscr, vscr, True)
      return 0
    lax.fori_loop(0, NV, vbody, 0, unroll=False)

    pltpu.sync_copy(ev, e_o.at[pl.ds(base, EWP)])
    pltpu.sync_copy(mloc, m_o.at[w])

  def make_merge(is_max):
    @functools.partial(
        pl.kernel,
        out_type=jax.ShapeDtypeStruct((NP,), f32),
        mesh=mesh,
        compiler_params=pltpu.CompilerParams(needs_layout_passes=False),
        scratch_types=[
            pltpu.VMEM((NW, SL), f32),
            pltpu.VMEM((SL,), f32),
        ])
    def kb(all_h, glob_o, buf, mg):
      w = _wid()
      pltpu.sync_copy(all_h.at[:, pl.ds(w * SL, SL)], buf)

      def cbody(k, _):
        off = k * L
        acc = buf[0, pl.ds(off, L)]
        for r in range(1, NW):
          v = buf[r, pl.ds(off, L)]
          acc = jnp.maximum(acc, v) if is_max else acc + v
        mg[pl.ds(off, L)] = acc
        return 0
      lax.fori_loop(0, SL // L, cbody, 0, unroll=False)
      pltpu.sync_copy(mg, glob_o.at[pl.ds(w * SL, SL)])
    return kb

  kb_max = make_merge(True)
  kb_sum = make_merge(False)

  @functools.partial(
      pl.kernel,
      out_type=(jax.ShapeDtypeStruct((EP,), f32),
                jax.ShapeDtypeStruct((NW, NP), f32)),
      mesh=mesh,
      compiler_params=pltpu.CompilerParams(needs_layout_passes=False),
      scratch_types=[
          pltpu.VMEM((EWP,), i32),
          pltpu.VMEM((EWP,), f32),
          pltpu.VMEM((EWP,), f32),
          pltpu.VMEM((NP,), f32),
          pltpu.VMEM((L,), i32),
          pltpu.VMEM((L,), f32),
          pltpu.SemaphoreType.DMA,
      ])
  def kc(e_h, h_h, mg_h, ee_o, s_o, hv, ev, mv, sloc, kscr, vscr, sem):
    w = _wid()
    base = w * EWP
    pltpu.sync_copy(h_h.at[pl.ds(base, EWP)], hv)
    pltpu.sync_copy(e_h.at[pl.ds(base, EWP)], ev)
    _chunk_gather(mg_h, hv, mv, sem, NCH)

    def zbody(i, _):
      sloc[pl.ds(i * L, L)] = jnp.zeros((L,), f32)
      return 0
    lax.fori_loop(0, NP // L, zbody, 0, unroll=False)

    def vbody(i, _):
      off = i * L
      ex = jnp.exp(ev[pl.ds(off, L)] - mv[pl.ds(off, L)])
      ev[pl.ds(off, L)] = ex
      _seg_rmw(hv[pl.ds(off, L)], ex, sloc, kscr, vscr, False)
      return 0
    lax.fori_loop(0, NV, vbody, 0, unroll=False)

    pltpu.sync_copy(ev, ee_o.at[pl.ds(base, EWP)])
    pltpu.sync_copy(sloc, s_o.at[w])

  @functools.partial(
      pl.kernel,
      out_type=jax.ShapeDtypeStruct((NC, NPO, D), f32),
      mesh=mesh,
      compiler_params=pltpu.CompilerParams(needs_layout_passes=False),
      scratch_types=[
          pltpu.VMEM((NCH, CH), i32),
          pltpu.VMEM((NCH, CH), i32),
          pltpu.VMEM((CH,), f32),
          pltpu.VMEM((CH,), f32),
          pltpu.VMEM((CH,), f32),
          pltpu.VMEM((CH, D), f32),
          pltpu.VMEM_SHARED((NPO, D), f32),
      ])
  def ke(ee_h, h3_h, t3_h, sg_h, emb_h, zeros_h, out_o,
         h2d, t2d, eev, sv, av, rows, out_sh):
    c = lax.axis_index("c")
    s = lax.axis_index("s")
    w = s * NC + c
    pltpu.sync_copy(h3_h.at[w], h2d)
    pltpu.sync_copy(t3_h.at[w], t2d)
    pltpu.sync_copy(zeros_h.at[pl.ds(s * SLC, SLC)],
                    out_sh.at[pl.ds(s * SLC, SLC)])
    plsc.subcore_barrier()

    def cbody(j, _):
      pltpu.sync_copy(ee_h.at[pl.ds(w * EWP + j * CH, CH)], eev)
      pltpu.sync_copy(sg_h.at[h2d.at[j]], sv)
      for kk in range(CH // L):
        at16 = eev[pl.ds(kk * L, L)] / (sv[pl.ds(kk * L, L)] + 1e-16)
        av[pl.ds(kk * L, L)] = at16
      pltpu.sync_copy(emb_h.at[t2d.at[j]], rows)
      for g in range(CH // L):
        at16 = av[pl.ds(g * L, L)]
        for r16 in range(L):
          r = g * L + r16
          a = at16[r16]
          for kk in range(D // L):
            rows[r, pl.ds(kk * L, L)] = rows[r, pl.ds(kk * L, L)] * a
      pltpu.sync_copy(rows, out_sh.at[h2d.at[j]], add=True)
      return 0
    lax.fori_loop(0, NCH, cbody, 0, unroll=False)

    plsc.subcore_barrier()
    pltpu.sync_copy(out_sh.at[pl.ds(s * SLC, SLC)],
                    out_o.at[c, pl.ds(s * SLC, SLC)])

  def tc0_body(emb_ref, w_ref, rel_ref, wr_ref, ab_ref):
    dn = (((1,), (1,)), ((), ()))
    wra = lax.dot_general(w_ref[0:D, :], rel_ref[...], dn,
                          preferred_element_type=f32, precision=lax.Precision.HIGHEST)
    wrb = lax.dot_general(w_ref[D:2 * D, :], rel_ref[...], dn,
                          preferred_element_type=f32, precision=lax.Precision.HIGHEST)
    wr = jnp.concatenate([wra, wrb], axis=1)
    wr_ref[...] = wr
    ab_ref[...] = jnp.dot(emb_ref[...], wr, preferred_element_type=f32, precision=lax.Precision.HIGHEST)

  tc0 = pl.pallas_call(
      tc0_body,
      out_shape=(jax.ShapeDtypeStruct((D, 2 * R), f32),
                 jax.ShapeDtypeStruct((N, 2 * R), f32)))

  BN = 1000 if N % 1000 == 0 else N

  def ktc_body(outp_ref, emb_ref, res_ref, wr_ref,
               embn_ref, resn_ref, ab_ref):
    x = outp_ref[0] + outp_ref[1] + emb_ref[...]
    n2 = jnp.sum(x * x, axis=-1, keepdims=True)
    y = x / jnp.maximum(jnp.sqrt(n2), 1e-12)
    embn_ref[...] = y
    resn_ref[...] = res_lambda * res_ref[...] + y
    ab_ref[...] = jnp.dot(y, wr_ref[...], preferred_element_type=f32, precision=lax.Precision.HIGHEST)

  ktc = pl.pallas_call(
      ktc_body,
      grid=(N // BN,),
      in_specs=[
          pl.BlockSpec((2, BN, D), lambda i: (0, i, 0)),
          pl.BlockSpec((BN, D), lambda i: (i, 0)),
          pl.BlockSpec((BN, D), lambda i: (i, 0)),
          pl.BlockSpec((D, 2 * R), lambda i: (0, 0)),
      ],
      out_specs=[
          pl.BlockSpec((BN, D), lambda i: (i, 0)),
          pl.BlockSpec((BN, D), lambda i: (i, 0)),
          pl.BlockSpec((BN, 2 * R), lambda i: (i, 0)),
      ],
      out_shape=(jax.ShapeDtypeStruct((N, D), f32),
                 jax.ShapeDtypeStruct((N, D), f32),
                 jax.ShapeDtypeStruct((N, 2 * R), f32)))

  h = edge_index[0]
  t = edge_index[1]
  et = edge_type
  pad = EP - E
  hp = jnp.concatenate([h, jnp.full((pad,), N, i32)])
  tp = jnp.concatenate([t, jnp.zeros((pad,), i32)])
  etp = jnp.concatenate([et, jnp.zeros((pad,), i32)])
  hcl = jnp.minimum(hp, N - 1)
  tcl = jnp.minimum(tp, N - 1)
  idx1 = hcl * (2 * R) + etp
  idx2 = tcl * (2 * R) + R + etp
  h3 = hp.reshape(NW, NCH, CH)
  t3 = tp.reshape(NW, NCH, CH)
  zeros_np = jnp.zeros((NPO, D), f32)

  wr, ab = tc0(entity_emb, W, relation_emb)
  emb = entity_emb
  res = entity_emb
  for _ in range(n_hops):
    abf = ab.reshape(N * 2 * R)
    e_all, m_all = ka(idx1, idx2, hp, abf)
    m_glob = kb_max(m_all)
    ee, s_all = kc(e_all, hp, m_glob)
    s_glob = kb_sum(s_all)
    outp = ke(ee, h3, t3, s_glob, emb, zeros_np)
    emb, res, ab = ktc(outp[:, :N, :], emb, res, wr)
  return res


# speedup vs baseline: 8.5068x; 1.2512x over previous
import functools

import jax
import jax.numpy as jnp
from jax import lax
from jax.experimental import pallas as pl
from jax.experimental.pallas import tpu as pltpu
from jax.experimental.pallas import tpu_sc as plsc

NC = 2
NS = 16
NW = NC * NS
L = 16
CH = 128
NEG = -3.38e38


def _iota():
  return lax.iota(jnp.int32, L)


def _seg_rmw(h16, v16, acc_ref, kscr, vscr, is_max):
  h_s, v_s = plsc.sort_key_val(h16, v16)
  ii = _iota()
  kscr[...] = h_s
  h_prev = plsc.load_gather(kscr, [jnp.maximum(ii - 1, 0)])
  is_start = (ii == 0) | (h_s != h_prev)
  rs = plsc.cummax(jnp.where(is_start, ii, -1))
  m = v_s
  for k in (1, 2, 4, 8):
    vscr[...] = m
    prev = plsc.load_gather(vscr, [jnp.maximum(ii - k, 0)])
    valid = (ii - k) >= rs
    if is_max:
      m = jnp.where(valid, jnp.maximum(m, prev), m)
    else:
      m = jnp.where(valid, m + prev, m)
  h_next = plsc.load_gather(kscr, [jnp.minimum(ii + 1, L - 1)])
  is_last = (ii == L - 1) | (h_s != h_next)
  cur = plsc.load_gather(acc_ref, [h_s])
  upd = jnp.maximum(cur, m) if is_max else cur + m
  plsc.store_scatter(acc_ref, [h_s], upd, mask=is_last)


def _chunk_gather(src_hbm, idx_ref, dst_ref, sem, nch):
  G = 8

  def gbody(g, _):
    descs = []
    for jj in range(G):
      off = (g * G + jj) * CH
      descs.append(
          pltpu.async_copy(
              src_hbm.at[idx_ref.at[pl.ds(off, CH)]],
              dst_ref.at[pl.ds(off, CH)], sem))
    for d in descs:
      d.wait()
    return 0

  lax.fori_loop(0, nch // G, gbody, 0, unroll=False)


def _wid():
  return lax.axis_index("s") * NC + lax.axis_index("c")


def kernel(entity_emb, relation_emb, W, edge_index, edge_type):
  N, D = entity_emb.shape
  R = relation_emb.shape[0]
  E = edge_index.shape[1]
  n_hops = 2
  res_lambda = 0.5

  NCH = -(-E // (NW * CH))
  NCH = -(-NCH // 8) * 8
  EWP = NCH * CH
  EP = NW * EWP
  NP = -(-(N + 1) // (NW * CH)) * (NW * CH)
  SL = NP // NW
  NPO = -(-(N + 1) // CH) * CH
  SLC = NPO // NS
  NV = EWP // L

  mesh = plsc.VectorSubcoreMesh(
      core_axis_name="c", subcore_axis_name="s",
      num_cores=NC, num_subcores=NS)
  f32 = jnp.float32
  i32 = jnp.int32

  @functools.partial(
      pl.kernel,
      out_type=(jax.ShapeDtypeStruct((EP,), f32),
                jax.ShapeDtypeStruct((NW, NP), f32)),
      mesh=mesh,
      compiler_params=pltpu.CompilerParams(needs_layout_passes=False),
      scratch_types=[
          pltpu.VMEM((EWP,), i32),
          pltpu.VMEM((EWP,), f32),
          pltpu.VMEM((EWP,), f32),
          pltpu.VMEM((EWP,), i32),
          pltpu.VMEM((EWP,), f32),
          pltpu.VMEM((NP,), f32),
          pltpu.VMEM((L,), i32),
          pltpu.VMEM((L,), f32),
          pltpu.SemaphoreType.DMA,
      ])
  def ka(idx1_h, idx2_h, h_h, ab_h, e_o, m_o,
         idxb, va, vb, hv, ev, mloc, kscr, vscr, sem):
    w = _wid()
    base = w * EWP
    pltpu.sync_copy(idx1_h.at[pl.ds(base, EWP)], idxb)
    _chunk_gather(ab_h, idxb, va, sem, NCH)
    pltpu.sync_copy(idx2_h.at[pl.ds(base, EWP)], idxb)
    _chunk_gather(ab_h, idxb, vb, sem, NCH)
    pltpu.sync_copy(h_h.at[pl.ds(base, EWP)], hv)

    def zbody(i, _):
      mloc[pl.ds(i * L, L)] = jnp.full((L,), NEG, f32)
      return 0
    lax.fori_loop(0, NP // L, zbody, 0, unroll=False)

    def vbody(i, _):
      off = i * L
      x = va[pl.ds(off, L)] + vb[pl.ds(off, L)]
      e16 = jnp.where(x > 0, x, 0.2 * x)
      ev[pl.ds(off, L)] = e16
      _seg_rmw(hv[pl.ds(off, L)], e16, mloc, kscr, vscr, True)
      return 0
    lax.fori_loop(0, NV, vbody, 0, unroll=False)

    pltpu.sync_copy(ev, e_o.at[pl.ds(base, EWP)])
    pltpu.sync_copy(mloc, m_o.at[w])

  def make_merge(is_max):
    @functools.partial(
        pl.kernel,
        out_type=jax.ShapeDtypeStruct((NP,), f32),
        mesh=mesh,
        compiler_params=pltpu.CompilerParams(needs_layout_passes=False),
        scratch_types=[
            pltpu.VMEM((NW, SL), f32),
            pltpu.VMEM((SL,), f32),
        ])
    def kb(all_h, glob_o, buf, mg):
      w = _wid()
      pltpu.sync_copy(all_h.at[:, pl.ds(w * SL, SL)], buf)

      def cbody(k, _):
        off = k * L
        acc = buf[0, pl.ds(off, L)]
        for r in range(1, NW):
          v = buf[r, pl.ds(off, L)]
          acc = jnp.maximum(acc, v) if is_max else acc + v
        mg[pl.ds(off, L)] = acc
        return 0
      lax.fori_loop(0, SL // L, cbody, 0, unroll=False)
      pltpu.sync_copy(mg, glob_o.at[pl.ds(w * SL, SL)])
    return kb

  kb_max = make_merge(True)
  kb_sum = make_merge(False)

  @functools.partial(
      pl.kernel,
      out_type=(jax.ShapeDtypeStruct((EP,), f32),
                jax.ShapeDtypeStruct((NW, NP), f32)),
      mesh=mesh,
      compiler_params=pltpu.CompilerParams(needs_layout_passes=False),
      scratch_types=[
          pltpu.VMEM((EWP,), i32),
          pltpu.VMEM((EWP,), f32),
          pltpu.VMEM((EWP,), f32),
          pltpu.VMEM((NP,), f32),
          pltpu.VMEM((L,), i32),
          pltpu.VMEM((L,), f32),
          pltpu.SemaphoreType.DMA,
      ])
  def kc(e_h, h_h, mg_h, ee_o, s_o, hv, ev, mv, sloc, kscr, vscr, sem):
    w = _wid()
    base = w * EWP
    pltpu.sync_copy(h_h.at[pl.ds(base, EWP)], hv)
    pltpu.sync_copy(e_h.at[pl.ds(base, EWP)], ev)
    _chunk_gather(mg_h, hv, mv, sem, NCH)

    def zbody(i, _):
      sloc[pl.ds(i * L, L)] = jnp.zeros((L,), f32)
      return 0
    lax.fori_loop(0, NP // L, zbody, 0, unroll=False)

    def vbody(i, _):
      off = i * L
      ex = jnp.exp(ev[pl.ds(off, L)] - mv[pl.ds(off, L)])
      ev[pl.ds(off, L)] = ex
      _seg_rmw(hv[pl.ds(off, L)], ex, sloc, kscr, vscr, False)
      return 0
    lax.fori_loop(0, NV, vbody, 0, unroll=False)

    pltpu.sync_copy(ev, ee_o.at[pl.ds(base, EWP)])
    pltpu.sync_copy(sloc, s_o.at[w])

  CE = 64
  NCHE = EWP // CE
  NB = 4

  @functools.partial(
      pl.kernel,
      out_type=jax.ShapeDtypeStruct((NC, NPO, D), f32),
      mesh=mesh,
      compiler_params=pltpu.CompilerParams(needs_layout_passes=False),
      scratch_types=[
          pltpu.VMEM((NP,), f32),
          [pltpu.VMEM((2, CE), i32)] * NB,
          [pltpu.VMEM((CE,), i32)] * NB,
          [pltpu.VMEM((CE,), f32)] * NB,
          [pltpu.VMEM((CE, D), f32)] * NB,
          pltpu.VMEM_SHARED((NPO, D), f32),
          [pltpu.SemaphoreType.DMA] * NB,
          [pltpu.SemaphoreType.DMA] * NB,
          [pltpu.SemaphoreType.DMA] * NB,
      ])
  def ke(ee_h, ht_h, sg_h, emb_h, zeros_h, out_o,
         sgv, htq, hsc, eev, rows, out_sh, sem_i, sem_g, sem_s):
    c = lax.axis_index("c")
    s = lax.axis_index("s")
    w = s * NC + c
    pltpu.sync_copy(sg_h, sgv)
    pltpu.sync_copy(zeros_h.at[pl.ds(s * SLC, SLC)],
                    out_sh.at[pl.ds(s * SLC, SLC)])
    plsc.subcore_barrier()

    def fire_idx(j, b):
      pltpu.async_copy(ht_h.at[w, j], htq[b], sem_i[b])
      pltpu.async_copy(ee_h.at[pl.ds(w * EWP + j * CE, CE)], eev[b], sem_i[b])

    def wait_idx(b):
      pltpu.make_async_copy(ht_h.at[0, 0], htq[b], sem_i[b]).wait()
      pltpu.make_async_copy(ee_h.at[pl.ds(0, CE)], eev[b], sem_i[b]).wait()

    def fire_rows(j, b):
      pltpu.async_copy(emb_h.at[htq[b].at[1]], rows[b], sem_g[b])

    def wait_rows(b):
      pltpu.make_async_copy(emb_h.at[pl.ds(0, CE)], rows[b], sem_g[b]).wait()

    def wait_scat(b):
      pltpu.make_async_copy(emb_h.at[pl.ds(0, CE)], rows[b], sem_s[b]).wait()

    fire_idx(0, 0)
    wait_idx(0)
    fire_rows(0, 0)
    fire_idx(1, 1)

    def cbody(g, _):
      for b in range(NB):
        j = g * NB + b
        b1 = (b + 1) % NB
        b2 = (b + 2) % NB

        @pl.when(j + 1 < NCHE)
        def _():
          wait_idx(b1)

        @pl.when(j >= NB - 1)
        def _():
          wait_scat(b1)

        @pl.when(j + 1 < NCHE)
        def _():
          fire_rows(j + 1, b1)

        @pl.when(j + 2 < NCHE)
        def _():
          fire_idx(j + 2, b2)

        wait_rows(b)

        ats = []
        for gg in range(CE // L):
          h16 = htq[b][0, pl.ds(gg * L, L)]
          s16 = plsc.load_gather(sgv, [h16])
          at16 = eev[b][pl.ds(gg * L, L)] / (s16 + 1e-16)
          ats.append(at16)
          hsc[b][pl.ds(gg * L, L)] = h16

        for gg in range(CE // L):
          at16 = ats[gg]
          for r16 in range(L):
            r = gg * L + r16
            a = at16[r16]
            for kk in range(D // L):
              rows[b][r, pl.ds(kk * L, L)] = rows[b][r, pl.ds(kk * L, L)] * a

        pltpu.async_copy(rows[b], out_sh.at[hsc[b]], sem_s[b], add=True)
      return 0
    lax.fori_loop(0, NCHE // NB, cbody, 0, unroll=False)

    for j in range(NCHE - (NB - 1), NCHE):
      wait_scat(j % NB)

    plsc.subcore_barrier()
    pltpu.sync_copy(out_sh.at[pl.ds(s * SLC, SLC)],
                    out_o.at[c, pl.ds(s * SLC, SLC)])

  def tc0_body(emb_ref, w_ref, rel_ref, wr_ref, ab_ref):
    dn = (((1,), (1,)), ((), ()))
    wra = lax.dot_general(w_ref[0:D, :], rel_ref[...], dn,
                          preferred_element_type=f32, precision=lax.Precision.HIGHEST)
    wrb = lax.dot_general(w_ref[D:2 * D, :], rel_ref[...], dn,
                          preferred_element_type=f32, precision=lax.Precision.HIGHEST)
    wr = jnp.concatenate([wra, wrb], axis=1)
    wr_ref[...] = wr
    ab_ref[...] = jnp.dot(emb_ref[...], wr, preferred_element_type=f32, precision=lax.Precision.HIGHEST)

  tc0 = pl.pallas_call(
      tc0_body,
      out_shape=(jax.ShapeDtypeStruct((D, 2 * R), f32),
                 jax.ShapeDtypeStruct((N, 2 * R), f32)))

  BN = 1000 if N % 1000 == 0 else N

  def ktc_body(outp_ref, emb_ref, res_ref, wr_ref,
               embn_ref, resn_ref, ab_ref):
    x = outp_ref[0] + outp_ref[1] + emb_ref[...]
    n2 = jnp.sum(x * x, axis=-1, keepdims=True)
    y = x / jnp.maximum(jnp.sqrt(n2), 1e-12)
    embn_ref[...] = y
    resn_ref[...] = res_lambda * res_ref[...] + y
    ab_ref[...] = jnp.dot(y, wr_ref[...], preferred_element_type=f32, precision=lax.Precision.HIGHEST)

  ktc = pl.pallas_call(
      ktc_body,
      grid=(N // BN,),
      in_specs=[
          pl.BlockSpec((2, BN, D), lambda i: (0, i, 0)),
          pl.BlockSpec((BN, D), lambda i: (i, 0)),
          pl.BlockSpec((BN, D), lambda i: (i, 0)),
          pl.BlockSpec((D, 2 * R), lambda i: (0, 0)),
      ],
      out_specs=[
          pl.BlockSpec((BN, D), lambda i: (i, 0)),
          pl.BlockSpec((BN, D), lambda i: (i, 0)),
          pl.BlockSpec((BN, 2 * R), lambda i: (i, 0)),
      ],
      out_shape=(jax.ShapeDtypeStruct((N, D), f32),
                 jax.ShapeDtypeStruct((N, D), f32),
                 jax.ShapeDtypeStruct((N, 2 * R), f32)))

  h = edge_index[0]
  t = edge_index[1]
  et = edge_type
  pad = EP - E
  hp = jnp.concatenate([h, jnp.full((pad,), N, i32)])
  tp = jnp.concatenate([t, jnp.zeros((pad,), i32)])
  etp = jnp.concatenate([et, jnp.zeros((pad,), i32)])
  hcl = jnp.minimum(hp, N - 1)
  tcl = jnp.minimum(tp, N - 1)
  idx1 = hcl * (2 * R) + etp
  idx2 = tcl * (2 * R) + R + etp
  ht3 = jnp.stack([hp.reshape(NW, NCHE, CE), tp.reshape(NW, NCHE, CE)],
                  axis=2)
  zeros_np = jnp.zeros((NPO, D), f32)

  wr, ab = tc0(entity_emb, W, relation_emb)
  emb = entity_emb
  res = entity_emb
  for _ in range(n_hops):
    abf = ab.reshape(N * 2 * R)
    e_all, m_all = ka(idx1, idx2, hp, abf)
    m_glob = kb_max(m_all)
    ee, s_all = kc(e_all, hp, m_glob)
    s_glob = kb_sum(s_all)
    outp = ke(ee, ht3, s_glob, emb, zeros_np)
    emb, res, ab = ktc(outp[:, :N, :], emb, res, wr)
  return res


# speedup vs baseline: 8.8703x; 1.0427x over previous
import functools

import jax
import jax.numpy as jnp
from jax import lax
from jax.experimental import pallas as pl
from jax.experimental.pallas import tpu as pltpu
from jax.experimental.pallas import tpu_sc as plsc

NC = 2
NS = 16
NW = NC * NS
L = 16
CH = 128
NEG = -3.38e38


def _iota():
  return lax.iota(jnp.int32, L)


def _seg_rmw(h16, v16, acc_ref, kscr, vscr, is_max):
  h_s, v_s = plsc.sort_key_val(h16, v16)
  ii = _iota()
  kscr[...] = h_s
  h_prev = plsc.load_gather(kscr, [jnp.maximum(ii - 1, 0)])
  is_start = (ii == 0) | (h_s != h_prev)
  rs = plsc.cummax(jnp.where(is_start, ii, -1))
  m = v_s
  for k in (1, 2, 4, 8):
    vscr[...] = m
    prev = plsc.load_gather(vscr, [jnp.maximum(ii - k, 0)])
    valid = (ii - k) >= rs
    if is_max:
      m = jnp.where(valid, jnp.maximum(m, prev), m)
    else:
      m = jnp.where(valid, m + prev, m)
  h_next = plsc.load_gather(kscr, [jnp.minimum(ii + 1, L - 1)])
  is_last = (ii == L - 1) | (h_s != h_next)
  cur = plsc.load_gather(acc_ref, [h_s])
  upd = jnp.maximum(cur, m) if is_max else cur + m
  plsc.store_scatter(acc_ref, [h_s], upd, mask=is_last)


def _chunk_gather(src_hbm, idx_ref, dst_ref, sem, nch):
  G = 8

  def gbody(g, _):
    descs = []
    for jj in range(G):
      off = (g * G + jj) * CH
      descs.append(
          pltpu.async_copy(
              src_hbm.at[idx_ref.at[pl.ds(off, CH)]],
              dst_ref.at[pl.ds(off, CH)], sem))
    for d in descs:
      d.wait()
    return 0

  lax.fori_loop(0, nch // G, gbody, 0, unroll=False)


def _wid():
  return lax.axis_index("s") * NC + lax.axis_index("c")


def kernel(entity_emb, relation_emb, W, edge_index, edge_type):
  N, D = entity_emb.shape
  R = relation_emb.shape[0]
  E = edge_index.shape[1]
  n_hops = 2
  res_lambda = 0.5

  NCH = -(-E // (NW * CH))
  NCH = -(-NCH // 8) * 8
  EWP = NCH * CH
  EP = NW * EWP
  NP = -(-(N + 1) // (NW * CH)) * (NW * CH)
  SL = NP // NW
  NPO = -(-(N + 1) // CH) * CH
  SLC = NPO // NS
  NV = EWP // L

  mesh = plsc.VectorSubcoreMesh(
      core_axis_name="c", subcore_axis_name="s",
      num_cores=NC, num_subcores=NS)
  f32 = jnp.float32
  i32 = jnp.int32

  @functools.partial(
      pl.kernel,
      out_type=(jax.ShapeDtypeStruct((EP,), f32),
                jax.ShapeDtypeStruct((NW, NP), f32)),
      mesh=mesh,
      compiler_params=pltpu.CompilerParams(needs_layout_passes=False),
      scratch_types=[
          pltpu.VMEM((EWP,), i32),
          pltpu.VMEM((EWP,), f32),
          pltpu.VMEM((EWP,), f32),
          pltpu.VMEM((EWP,), i32),
          pltpu.VMEM((EWP,), f32),
          pltpu.VMEM((NP,), f32),
          pltpu.VMEM((L,), i32),
          pltpu.VMEM((L,), f32),
          pltpu.SemaphoreType.DMA,
      ])
  def ka(idx1_h, idx2_h, h_h, ab_h, e_o, m_o,
         idxb, va, vb, hv, ev, mloc, kscr, vscr, sem):
    w = _wid()
    base = w * EWP
    pltpu.sync_copy(idx1_h.at[pl.ds(base, EWP)], idxb)
    _chunk_gather(ab_h, idxb, va, sem, NCH)
    pltpu.sync_copy(idx2_h.at[pl.ds(base, EWP)], idxb)
    _chunk_gather(ab_h, idxb, vb, sem, NCH)
    pltpu.sync_copy(h_h.at[pl.ds(base, EWP)], hv)

    def zbody(i, _):
      mloc[pl.ds(i * L, L)] = jnp.full((L,), NEG, f32)
      return 0
    lax.fori_loop(0, NP // L, zbody, 0, unroll=False)

    def vbody(i, _):
      off = i * L
      x = va[pl.ds(off, L)] + vb[pl.ds(off, L)]
      e16 = jnp.where(x > 0, x, 0.2 * x)
      ev[pl.ds(off, L)] = e16
      _seg_rmw(hv[pl.ds(off, L)], e16, mloc, kscr, vscr, True)
      return 0
    lax.fori_loop(0, NV, vbody, 0, unroll=False)

    pltpu.sync_copy(ev, e_o.at[pl.ds(base, EWP)])
    pltpu.sync_copy(mloc, m_o.at[w])

  NPB = NP // CH

  def make_merge(is_max):
    def body(all_ref, out_ref):
      x = all_ref[...]
      out_ref[...] = jnp.max(x, axis=0) if is_max else jnp.sum(x, axis=0)
    return pl.pallas_call(
        body, out_shape=jax.ShapeDtypeStruct((NPB, CH), f32))

  kb_max = make_merge(True)
  kb_sum = make_merge(False)

  @functools.partial(
      pl.kernel,
      out_type=(jax.ShapeDtypeStruct((EP,), f32),
                jax.ShapeDtypeStruct((NW, NP), f32)),
      mesh=mesh,
      compiler_params=pltpu.CompilerParams(needs_layout_passes=False),
      scratch_types=[
          pltpu.VMEM((EWP,), i32),
          pltpu.VMEM((EWP,), f32),
          pltpu.VMEM((EWP,), f32),
          pltpu.VMEM((NP,), f32),
          pltpu.VMEM((L,), i32),
          pltpu.VMEM((L,), f32),
          pltpu.SemaphoreType.DMA,
      ])
  def kc(e_h, h_h, mg_h, ee_o, s_o, hv, ev, mv, sloc, kscr, vscr, sem):
    w = _wid()
    base = w * EWP
    pltpu.sync_copy(h_h.at[pl.ds(base, EWP)], hv)
    pltpu.sync_copy(e_h.at[pl.ds(base, EWP)], ev)
    _chunk_gather(mg_h, hv, mv, sem, NCH)

    def zbody(i, _):
      sloc[pl.ds(i * L, L)] = jnp.zeros((L,), f32)
      return 0
    lax.fori_loop(0, NP // L, zbody, 0, unroll=False)

    def vbody(i, _):
      off = i * L
      ex = jnp.exp(ev[pl.ds(off, L)] - mv[pl.ds(off, L)])
      ev[pl.ds(off, L)] = ex
      _seg_rmw(hv[pl.ds(off, L)], ex, sloc, kscr, vscr, False)
      return 0
    lax.fori_loop(0, NV, vbody, 0, unroll=False)

    pltpu.sync_copy(ev, ee_o.at[pl.ds(base, EWP)])
    pltpu.sync_copy(sloc, s_o.at[w])

  CE = 128
  NCHE = EWP // CE
  NB = 2

  @functools.partial(
      pl.kernel,
      out_type=jax.ShapeDtypeStruct((NC, NPO, D), f32),
      mesh=mesh,
      compiler_params=pltpu.CompilerParams(needs_layout_passes=False),
      scratch_types=[
          [pltpu.VMEM((2, CE), i32)] * NB,
          [pltpu.VMEM((CE,), i32)] * NB,
          [pltpu.VMEM((CE,), f32)] * NB,
          [pltpu.VMEM((CE, D), f32)] * NB,
          pltpu.VMEM_SHARED((NPO, D), f32),
          [pltpu.SemaphoreType.DMA] * NB,
          [pltpu.SemaphoreType.DMA] * NB,
          [pltpu.SemaphoreType.DMA] * NB,
      ])
  def ke(ee_h, ht_h, emb_h, zeros_h, out_o,
         htq, hsc, eev, rows, out_sh, sem_i, sem_g, sem_s):
    c = lax.axis_index("c")
    s = lax.axis_index("s")
    w = s * NC + c
    pltpu.sync_copy(zeros_h.at[pl.ds(s * SLC, SLC)],
                    out_sh.at[pl.ds(s * SLC, SLC)])
    plsc.subcore_barrier()

    def fire_idx(j, b):
      pltpu.async_copy(ht_h.at[w, j], htq[b], sem_i[b])
      pltpu.async_copy(ee_h.at[pl.ds(w * EWP + j * CE, CE)], eev[b], sem_i[b])

    def wait_idx(b):
      pltpu.make_async_copy(ht_h.at[0, 0], htq[b], sem_i[b]).wait()
      pltpu.make_async_copy(ee_h.at[pl.ds(0, CE)], eev[b], sem_i[b]).wait()

    def fire_rows(j, b):
      pltpu.async_copy(emb_h.at[htq[b].at[1]], rows[b], sem_g[b])

    def wait_rows(b):
      pltpu.make_async_copy(emb_h.at[pl.ds(0, CE)], rows[b], sem_g[b]).wait()

    def wait_scat(b):
      pltpu.make_async_copy(emb_h.at[pl.ds(0, CE)], rows[b], sem_s[b]).wait()

    fire_idx(0, 0)
    wait_idx(0)
    fire_rows(0, 0)
    fire_idx(1, 1)

    def cbody(g, _):
      for b in range(NB):
        j = g * NB + b
        b1 = (b + 1) % NB

        @pl.when(j + 1 < NCHE)
        def _():
          wait_idx(b1)

        @pl.when(j >= 1)
        def _():
          wait_scat(b1)

        @pl.when(j + 1 < NCHE)
        def _():
          fire_rows(j + 1, b1)

        wait_rows(b)

        ats = []
        for gg in range(CE // L):
          at16 = eev[b][pl.ds(gg * L, L)]
          ats.append(at16)
          hsc[b][pl.ds(gg * L, L)] = htq[b][0, pl.ds(gg * L, L)]

        @pl.when(j + 2 < NCHE)
        def _():
          fire_idx(j + 2, b)

        for gg in range(CE // L):
          at16 = ats[gg]
          for r16 in range(L):
            r = gg * L + r16
            a = at16[r16]
            for kk in range(D // L):
              rows[b][r, pl.ds(kk * L, L)] = rows[b][r, pl.ds(kk * L, L)] * a

        pltpu.async_copy(rows[b], out_sh.at[hsc[b]], sem_s[b], add=True)
      return 0
    lax.fori_loop(0, NCHE // NB, cbody, 0, unroll=False)

    wait_scat((NCHE - 1) % NB)

    plsc.subcore_barrier()
    pltpu.sync_copy(out_sh.at[pl.ds(s * SLC, SLC)],
                    out_o.at[c, pl.ds(s * SLC, SLC)])

  def tc0_body(emb_ref, w_ref, rel_ref, wr_ref, ab_ref):
    dn = (((1,), (1,)), ((), ()))
    wra = lax.dot_general(w_ref[0:D, :], rel_ref[...], dn,
                          preferred_element_type=f32, precision=lax.Precision.HIGHEST)
    wrb = lax.dot_general(w_ref[D:2 * D, :], rel_ref[...], dn,
                          preferred_element_type=f32, precision=lax.Precision.HIGHEST)
    wr = jnp.concatenate([wra, wrb], axis=1)
    wr_ref[...] = wr
    ab_ref[...] = jnp.dot(emb_ref[...], wr, preferred_element_type=f32, precision=lax.Precision.HIGHEST)

  tc0 = pl.pallas_call(
      tc0_body,
      out_shape=(jax.ShapeDtypeStruct((D, 2 * R), f32),
                 jax.ShapeDtypeStruct((N, 2 * R), f32)))

  BN = 1000 if N % 1000 == 0 else N

  def ktc_body(outp_ref, s2_ref, emb_ref, res_ref, wr_ref,
               embn_ref, resn_ref, ab_ref):
    x = (outp_ref[0] + outp_ref[1]) / (s2_ref[...] + 1e-16) + emb_ref[...]
    n2 = jnp.sum(x * x, axis=-1, keepdims=True)
    y = x / jnp.maximum(jnp.sqrt(n2), 1e-12)
    embn_ref[...] = y
    resn_ref[...] = res_lambda * res_ref[...] + y
    ab_ref[...] = jnp.dot(y, wr_ref[...], preferred_element_type=f32, precision=lax.Precision.HIGHEST)

  ktc = pl.pallas_call(
      ktc_body,
      grid=(N // BN,),
      in_specs=[
          pl.BlockSpec((2, BN, D), lambda i: (0, i, 0)),
          pl.BlockSpec((BN, 1), lambda i: (i, 0)),
          pl.BlockSpec((BN, D), lambda i: (i, 0)),
          pl.BlockSpec((BN, D), lambda i: (i, 0)),
          pl.BlockSpec((D, 2 * R), lambda i: (0, 0)),
      ],
      out_specs=[
          pl.BlockSpec((BN, D), lambda i: (i, 0)),
          pl.BlockSpec((BN, D), lambda i: (i, 0)),
          pl.BlockSpec((BN, 2 * R), lambda i: (i, 0)),
      ],
      out_shape=(jax.ShapeDtypeStruct((N, D), f32),
                 jax.ShapeDtypeStruct((N, D), f32),
                 jax.ShapeDtypeStruct((N, 2 * R), f32)))

  h = edge_index[0]
  t = edge_index[1]
  et = edge_type
  pad = EP - E
  hp = jnp.concatenate([h, jnp.full((pad,), N, i32)])
  tp = jnp.concatenate([t, jnp.zeros((pad,), i32)])
  etp = jnp.concatenate([et, jnp.zeros((pad,), i32)])
  hcl = jnp.minimum(hp, N - 1)
  tcl = jnp.minimum(tp, N - 1)
  idx1 = hcl * (2 * R) + etp
  idx2 = tcl * (2 * R) + R + etp
  ht3 = jnp.stack([hp.reshape(NW, NCHE, CE), tp.reshape(NW, NCHE, CE)],
                  axis=2)
  zeros_np = jnp.zeros((NPO, D), f32)

  wr, ab = tc0(entity_emb, W, relation_emb)
  emb = entity_emb
  res = entity_emb
  for _ in range(n_hops):
    abf = ab.reshape(N * 2 * R)
    e_all, m_all = ka(idx1, idx2, hp, abf)
    m_glob = kb_max(m_all.reshape(NW, NPB, CH)).reshape(NP)
    ee, s_all = kc(e_all, hp, m_glob)
    s_glob = kb_sum(s_all.reshape(NW, NPB, CH)).reshape(NP)
    s2 = s_glob[:N, None]
    outp = ke(ee, ht3, emb, zeros_np)
    emb, res, ab = ktc(outp[:, :N, :], s2, emb, res, wr)
  return res


# speedup vs baseline: 9.8729x; 1.1130x over previous
import functools

import jax
import jax.numpy as jnp
from jax import lax
from jax.experimental import pallas as pl
from jax.experimental.pallas import tpu as pltpu
from jax.experimental.pallas import tpu_sc as plsc

NC = 2
NS = 16
NW = NC * NS
L = 16
CH = 128
NEG = -3.38e38


def _iota():
  return lax.iota(jnp.int32, L)


def _seg_rmw(h16, v16, acc_ref, kscr, vscr, is_max):
  h_s, v_s = plsc.sort_key_val(h16, v16)
  ii = _iota()
  kscr[...] = h_s
  h_prev = plsc.load_gather(kscr, [jnp.maximum(ii - 1, 0)])
  is_start = (ii == 0) | (h_s != h_prev)
  rs = plsc.cummax(jnp.where(is_start, ii, -1))
  m = v_s
  for k in (1, 2, 4, 8):
    vscr[...] = m
    prev = plsc.load_gather(vscr, [jnp.maximum(ii - k, 0)])
    valid = (ii - k) >= rs
    if is_max:
      m = jnp.where(valid, jnp.maximum(m, prev), m)
    else:
      m = jnp.where(valid, m + prev, m)
  h_next = plsc.load_gather(kscr, [jnp.minimum(ii + 1, L - 1)])
  is_last = (ii == L - 1) | (h_s != h_next)
  cur = plsc.load_gather(acc_ref, [h_s])
  upd = jnp.maximum(cur, m) if is_max else cur + m
  plsc.store_scatter(acc_ref, [h_s], upd, mask=is_last)


G = 8


def _fire_round(src_hbm, idx_ref, dst_ref, sem, g):
  for jj in range(G):
    off = (g * G + jj) * CH
    pltpu.async_copy(
        src_hbm.at[idx_ref.at[pl.ds(off, CH)]],
        dst_ref.at[pl.ds(off, CH)], sem)


def _drain_round(src_hbm, dst_ref, sem):
  for jj in range(G):
    pltpu.make_async_copy(
        src_hbm.at[pl.ds(0, CH)], dst_ref.at[pl.ds(0, CH)], sem).wait()


def _wid():
  return lax.axis_index("s") * NC + lax.axis_index("c")


def kernel(entity_emb, relation_emb, W, edge_index, edge_type):
  N, D = entity_emb.shape
  R = relation_emb.shape[0]
  E = edge_index.shape[1]
  n_hops = 2
  res_lambda = 0.5

  NCH = -(-E // (NW * CH))
  NCH = -(-NCH // 8) * 8
  EWP = NCH * CH
  EP = NW * EWP
  NP = -(-(N + 1) // (NW * CH)) * (NW * CH)
  SL = NP // NW
  NPO = -(-(N + 1) // CH) * CH
  SLC = NPO // NS
  NV = EWP // L

  mesh = plsc.VectorSubcoreMesh(
      core_axis_name="c", subcore_axis_name="s",
      num_cores=NC, num_subcores=NS)
  f32 = jnp.float32
  i32 = jnp.int32

  @functools.partial(
      pl.kernel,
      out_type=(jax.ShapeDtypeStruct((EP,), f32),
                jax.ShapeDtypeStruct((NW, NP), f32)),
      mesh=mesh,
      compiler_params=pltpu.CompilerParams(needs_layout_passes=False),
      scratch_types=[
          pltpu.VMEM((EWP,), i32),
          pltpu.VMEM((EWP,), i32),
          pltpu.VMEM((EWP,), f32),
          pltpu.VMEM((EWP,), f32),
          pltpu.VMEM((EWP,), i32),
          pltpu.VMEM((EWP,), f32),
          pltpu.VMEM((NP,), f32),
          pltpu.VMEM((L,), i32),
          pltpu.VMEM((L,), f32),
          pltpu.SemaphoreType.DMA,
          pltpu.SemaphoreType.DMA,
          pltpu.SemaphoreType.DMA,
      ])
  def ka(idx1_h, idx2_h, h_h, ab_h, e_o, m_o,
         idxb, idxb2, va, vb, hv, ev, mloc, kscr, vscr, sema, semb, semi):
    w = _wid()
    base = w * EWP
    NR = NCH // G
    pltpu.async_copy(idx1_h.at[pl.ds(base, EWP)], idxb, semi)
    pltpu.async_copy(idx2_h.at[pl.ds(base, EWP)], idxb2, semi)
    pltpu.async_copy(h_h.at[pl.ds(base, EWP)], hv, semi)

    def zbody(i, _):
      mloc[pl.ds(i * L, L)] = jnp.full((L,), NEG, f32)
      return 0
    lax.fori_loop(0, NP // L, zbody, 0, unroll=False)

    pltpu.make_async_copy(idx1_h.at[pl.ds(0, EWP)], idxb, semi).wait()
    pltpu.make_async_copy(idx1_h.at[pl.ds(0, EWP)], idxb2, semi).wait()
    pltpu.make_async_copy(idx1_h.at[pl.ds(0, EWP)], hv, semi).wait()

    _fire_round(ab_h, idxb, va, sema, 0)
    _fire_round(ab_h, idxb2, vb, semb, 0)

    def vbody(i, _):
      off = i * L
      x = va[pl.ds(off, L)] + vb[pl.ds(off, L)]
      e16 = jnp.where(x > 0, x, 0.2 * x)
      ev[pl.ds(off, L)] = e16
      _seg_rmw(hv[pl.ds(off, L)], e16, mloc, kscr, vscr, True)
      return 0

    NVR = G * CH // L

    def rbody(g, _):
      @pl.when(g + 1 < NR)
      def _():
        _fire_round(ab_h, idxb, va, sema, g + 1)
        _fire_round(ab_h, idxb2, vb, semb, g + 1)
      _drain_round(ab_h, va, sema)
      _drain_round(ab_h, vb, semb)
      lax.fori_loop(g * NVR, (g + 1) * NVR, vbody, 0, unroll=False)
      return 0
    lax.fori_loop(0, NR, rbody, 0, unroll=False)

    pltpu.sync_copy(ev, e_o.at[pl.ds(base, EWP)])
    pltpu.sync_copy(mloc, m_o.at[w])

  NPB = NP // CH

  def make_merge(is_max):
    def body(all_ref, out_ref):
      x = all_ref[...]
      out_ref[...] = jnp.max(x, axis=0) if is_max else jnp.sum(x, axis=0)
    return pl.pallas_call(
        body, out_shape=jax.ShapeDtypeStruct((NPB, CH), f32))

  kb_max = make_merge(True)
  kb_sum = make_merge(False)

  @functools.partial(
      pl.kernel,
      out_type=(jax.ShapeDtypeStruct((EP,), f32),
                jax.ShapeDtypeStruct((NW, NP), f32)),
      mesh=mesh,
      compiler_params=pltpu.CompilerParams(needs_layout_passes=False),
      scratch_types=[
          pltpu.VMEM((EWP,), i32),
          pltpu.VMEM((EWP,), f32),
          pltpu.VMEM((EWP,), f32),
          pltpu.VMEM((NP,), f32),
          pltpu.VMEM((L,), i32),
          pltpu.VMEM((L,), f32),
          pltpu.SemaphoreType.DMA,
          pltpu.SemaphoreType.DMA,
      ])
  def kc(e_h, h_h, mg_h, ee_o, s_o, hv, ev, mv, sloc, kscr, vscr, sem, semi):
    w = _wid()
    base = w * EWP
    NR = NCH // G
    pltpu.async_copy(h_h.at[pl.ds(base, EWP)], hv, semi)
    pltpu.async_copy(e_h.at[pl.ds(base, EWP)], ev, semi)

    def zbody(i, _):
      sloc[pl.ds(i * L, L)] = jnp.zeros((L,), f32)
      return 0
    lax.fori_loop(0, NP // L, zbody, 0, unroll=False)

    pltpu.make_async_copy(h_h.at[pl.ds(0, EWP)], hv, semi).wait()
    pltpu.make_async_copy(e_h.at[pl.ds(0, EWP)], ev, semi).wait()

    _fire_round(mg_h, hv, mv, sem, 0)

    def vbody(i, _):
      off = i * L
      ex = jnp.exp(ev[pl.ds(off, L)] - mv[pl.ds(off, L)])
      ev[pl.ds(off, L)] = ex
      _seg_rmw(hv[pl.ds(off, L)], ex, sloc, kscr, vscr, False)
      return 0

    NVR = G * CH // L

    def rbody(g, _):
      @pl.when(g + 1 < NR)
      def _():
        _fire_round(mg_h, hv, mv, sem, g + 1)
      _drain_round(mg_h, mv, sem)
      lax.fori_loop(g * NVR, (g + 1) * NVR, vbody, 0, unroll=False)
      return 0
    lax.fori_loop(0, NR, rbody, 0, unroll=False)

    pltpu.sync_copy(ev, ee_o.at[pl.ds(base, EWP)])
    pltpu.sync_copy(sloc, s_o.at[w])

  CE = 128
  NCHE = EWP // CE
  NB = 2

  @functools.partial(
      pl.kernel,
      out_type=jax.ShapeDtypeStruct((NC, NPO, D), f32),
      mesh=mesh,
      compiler_params=pltpu.CompilerParams(needs_layout_passes=False),
      scratch_types=[
          [pltpu.VMEM((2, CE), i32)] * NB,
          [pltpu.VMEM((CE,), i32)] * NB,
          [pltpu.VMEM((CE,), f32)] * NB,
          [pltpu.VMEM((CE, D), f32)] * NB,
          pltpu.VMEM_SHARED((NPO, D), f32),
          [pltpu.SemaphoreType.DMA] * NB,
          [pltpu.SemaphoreType.DMA] * NB,
          [pltpu.SemaphoreType.DMA] * NB,
      ])
  def ke(ee_h, ht_h, emb_h, zeros_h, out_o,
         htq, hsc, eev, rows, out_sh, sem_i, sem_g, sem_s):
    c = lax.axis_index("c")
    s = lax.axis_index("s")
    w = s * NC + c
    pltpu.sync_copy(zeros_h.at[pl.ds(s * SLC, SLC)],
                    out_sh.at[pl.ds(s * SLC, SLC)])
    plsc.subcore_barrier()

    def fire_idx(j, b):
      pltpu.async_copy(ht_h.at[w, j], htq[b], sem_i[b])
      pltpu.async_copy(ee_h.at[pl.ds(w * EWP + j * CE, CE)], eev[b], sem_i[b])

    def wait_idx(b):
      pltpu.make_async_copy(ht_h.at[0, 0], htq[b], sem_i[b]).wait()
      pltpu.make_async_copy(ee_h.at[pl.ds(0, CE)], eev[b], sem_i[b]).wait()

    def fire_rows(j, b):
      pltpu.async_copy(emb_h.at[htq[b].at[1]], rows[b], sem_g[b])

    def wait_rows(b):
      pltpu.make_async_copy(emb_h.at[pl.ds(0, CE)], rows[b], sem_g[b]).wait()

    def wait_scat(b):
      pltpu.make_async_copy(emb_h.at[pl.ds(0, CE)], rows[b], sem_s[b]).wait()

    fire_idx(0, 0)
    wait_idx(0)
    fire_rows(0, 0)
    fire_idx(1, 1)

    def cbody(g, _):
      for b in range(NB):
        j = g * NB + b
        b1 = (b + 1) % NB

        @pl.when(j + 1 < NCHE)
        def _():
          wait_idx(b1)

        @pl.when(j >= 1)
        def _():
          wait_scat(b1)

        @pl.when(j + 1 < NCHE)
        def _():
          fire_rows(j + 1, b1)

        wait_rows(b)

        ats = []
        for gg in range(CE // L):
          at16 = eev[b][pl.ds(gg * L, L)]
          ats.append(at16)
          hsc[b][pl.ds(gg * L, L)] = htq[b][0, pl.ds(gg * L, L)]

        @pl.when(j + 2 < NCHE)
        def _():
          fire_idx(j + 2, b)

        for gg in range(CE // L):
          at16 = ats[gg]
          for r16 in range(L):
            r = gg * L + r16
            a = at16[r16]
            for kk in range(D // L):
              rows[b][r, pl.ds(kk * L, L)] = rows[b][r, pl.ds(kk * L, L)] * a

        pltpu.async_copy(rows[b], out_sh.at[hsc[b]], sem_s[b], add=True)
      return 0
    lax.fori_loop(0, NCHE // NB, cbody, 0, unroll=False)

    wait_scat((NCHE - 1) % NB)

    plsc.subcore_barrier()
    pltpu.sync_copy(out_sh.at[pl.ds(s * SLC, SLC)],
                    out_o.at[c, pl.ds(s * SLC, SLC)])

  def tc0_body(emb_ref, w_ref, rel_ref, wr_ref, ab_ref):
    dn = (((1,), (1,)), ((), ()))
    wra = lax.dot_general(w_ref[0:D, :], rel_ref[...], dn,
                          preferred_element_type=f32, precision=lax.Precision.HIGHEST)
    wrb = lax.dot_general(w_ref[D:2 * D, :], rel_ref[...], dn,
                          preferred_element_type=f32, precision=lax.Precision.HIGHEST)
    wr = jnp.concatenate([wra, wrb], axis=1)
    wr_ref[...] = wr
    ab_ref[...] = jnp.dot(emb_ref[...], wr, preferred_element_type=f32, precision=lax.Precision.HIGHEST)

  tc0 = pl.pallas_call(
      tc0_body,
      out_shape=(jax.ShapeDtypeStruct((D, 2 * R), f32),
                 jax.ShapeDtypeStruct((N, 2 * R), f32)))

  BN = 1000 if N % 1000 == 0 else N

  def ktc_body(outp_ref, s2_ref, emb_ref, res_ref, wr_ref,
               embn_ref, resn_ref, ab_ref):
    x = (outp_ref[0] + outp_ref[1]) / (s2_ref[...] + 1e-16) + emb_ref[...]
    n2 = jnp.sum(x * x, axis=-1, keepdims=True)
    y = x / jnp.maximum(jnp.sqrt(n2), 1e-12)
    embn_ref[...] = y
    resn_ref[...] = res_lambda * res_ref[...] + y
    ab_ref[...] = jnp.dot(y, wr_ref[...], preferred_element_type=f32, precision=lax.Precision.HIGHEST)

  ktc = pl.pallas_call(
      ktc_body,
      grid=(N // BN,),
      in_specs=[
          pl.BlockSpec((2, BN, D), lambda i: (0, i, 0)),
          pl.BlockSpec((BN, 1), lambda i: (i, 0)),
          pl.BlockSpec((BN, D), lambda i: (i, 0)),
          pl.BlockSpec((BN, D), lambda i: (i, 0)),
          pl.BlockSpec((D, 2 * R), lambda i: (0, 0)),
      ],
      out_specs=[
          pl.BlockSpec((BN, D), lambda i: (i, 0)),
          pl.BlockSpec((BN, D), lambda i: (i, 0)),
          pl.BlockSpec((BN, 2 * R), lambda i: (i, 0)),
      ],
      out_shape=(jax.ShapeDtypeStruct((N, D), f32),
                 jax.ShapeDtypeStruct((N, D), f32),
                 jax.ShapeDtypeStruct((N, 2 * R), f32)))

  h = edge_index[0]
  t = edge_index[1]
  et = edge_type
  pad = EP - E
  hp = jnp.concatenate([h, jnp.full((pad,), N, i32)])
  tp = jnp.concatenate([t, jnp.zeros((pad,), i32)])
  etp = jnp.concatenate([et, jnp.zeros((pad,), i32)])
  hcl = jnp.minimum(hp, N - 1)
  tcl = jnp.minimum(tp, N - 1)
  idx1 = hcl * (2 * R) + etp
  idx2 = tcl * (2 * R) + R + etp
  ht3 = jnp.stack([hp.reshape(NW, NCHE, CE), tp.reshape(NW, NCHE, CE)],
                  axis=2)
  zeros_np = jnp.zeros((NPO, D), f32)

  wr, ab = tc0(entity_emb, W, relation_emb)
  emb = entity_emb
  res = entity_emb
  for _ in range(n_hops):
    abf = ab.reshape(N * 2 * R)
    e_all, m_all = ka(idx1, idx2, hp, abf)
    m_glob = kb_max(m_all.reshape(NW, NPB, CH)).reshape(NP)
    ee, s_all = kc(e_all, hp, m_glob)
    s_glob = kb_sum(s_all.reshape(NW, NPB, CH)).reshape(NP)
    s2 = s_glob[:N, None]
    outp = ke(ee, ht3, emb, zeros_np)
    emb, res, ab = ktc(outp[:, :N, :], s2, emb, res, wr)
  return res


# speedup vs baseline: 9.8831x; 1.0010x over previous
import functools

import jax
import jax.numpy as jnp
from jax import lax
from jax.experimental import pallas as pl
from jax.experimental.pallas import tpu as pltpu
from jax.experimental.pallas import tpu_sc as plsc

NC = 2
NS = 16
NW = NC * NS
L = 16
CH = 128
NEG = -3.38e38


def _iota():
  return lax.iota(jnp.int32, L)


def _seg_rmw(h16, v16, acc_ref, kscr, vscr, is_max):
  h_s, v_s = plsc.sort_key_val(h16, v16)
  ii = _iota()
  kscr[...] = h_s
  h_prev = plsc.load_gather(kscr, [jnp.maximum(ii - 1, 0)])
  is_start = (ii == 0) | (h_s != h_prev)
  rs = plsc.cummax(jnp.where(is_start, ii, -1))
  m = v_s
  for k in (1, 2, 4, 8):
    vscr[...] = m
    prev = plsc.load_gather(vscr, [jnp.maximum(ii - k, 0)])
    valid = (ii - k) >= rs
    if is_max:
      m = jnp.where(valid, jnp.maximum(m, prev), m)
    else:
      m = jnp.where(valid, m + prev, m)
  h_next = plsc.load_gather(kscr, [jnp.minimum(ii + 1, L - 1)])
  is_last = (ii == L - 1) | (h_s != h_next)
  cur = plsc.load_gather(acc_ref, [h_s])
  upd = jnp.maximum(cur, m) if is_max else cur + m
  plsc.store_scatter(acc_ref, [h_s], upd, mask=is_last)


G = 8


def _fire_round(src_hbm, idx_ref, dst_ref, sem, g):
  for jj in range(G):
    off = (g * G + jj) * CH
    pltpu.async_copy(
        src_hbm.at[idx_ref.at[pl.ds(off, CH)]],
        dst_ref.at[pl.ds(off, CH)], sem)


def _drain_round(src_hbm, dst_ref, sem):
  for jj in range(G):
    pltpu.make_async_copy(
        src_hbm.at[pl.ds(0, CH)], dst_ref.at[pl.ds(0, CH)], sem).wait()


def _wid():
  return lax.axis_index("s") * NC + lax.axis_index("c")


def kernel(entity_emb, relation_emb, W, edge_index, edge_type):
  N, D = entity_emb.shape
  R = relation_emb.shape[0]
  E = edge_index.shape[1]
  n_hops = 2
  res_lambda = 0.5

  NCH = -(-E // (NW * CH))
  NCH = -(-NCH // 8) * 8
  EWP = NCH * CH
  EP = NW * EWP
  NP = -(-(N + 1) // (NW * CH)) * (NW * CH)
  SL = NP // NW
  NPO = -(-(N + 1) // CH) * CH
  SLC = NPO // NS
  NV = EWP // L

  mesh = plsc.VectorSubcoreMesh(
      core_axis_name="c", subcore_axis_name="s",
      num_cores=NC, num_subcores=NS)
  f32 = jnp.float32
  i32 = jnp.int32

  @functools.partial(
      pl.kernel,
      out_type=(jax.ShapeDtypeStruct((EP,), f32),
                jax.ShapeDtypeStruct((NW, NP), f32)),
      mesh=mesh,
      compiler_params=pltpu.CompilerParams(needs_layout_passes=False),
      scratch_types=[
          pltpu.VMEM((EWP,), i32),
          pltpu.VMEM((EWP,), i32),
          pltpu.VMEM((EWP,), f32),
          pltpu.VMEM((EWP,), f32),
          pltpu.VMEM((EWP,), i32),
          pltpu.VMEM((EWP,), f32),
          pltpu.VMEM((NP,), f32),
          pltpu.VMEM((L,), i32),
          pltpu.VMEM((L,), f32),
          pltpu.SemaphoreType.DMA,
          pltpu.SemaphoreType.DMA,
          pltpu.SemaphoreType.DMA,
      ])
  def ka(idx1_h, idx2_h, h_h, ab_h, e_o, m_o,
         idxb, idxb2, va, vb, hv, ev, mloc, kscr, vscr, sema, semb, semi):
    w = _wid()
    base = w * EWP
    NR = NCH // G
    pltpu.async_copy(idx1_h.at[pl.ds(base, EWP)], idxb, semi)
    pltpu.async_copy(idx2_h.at[pl.ds(base, EWP)], idxb2, semi)
    pltpu.async_copy(h_h.at[pl.ds(base, EWP)], hv, semi)

    def zbody(i, _):
      mloc[pl.ds(i * L, L)] = jnp.full((L,), NEG, f32)
      return 0
    lax.fori_loop(0, NP // L, zbody, 0, unroll=False)

    pltpu.make_async_copy(idx1_h.at[pl.ds(0, EWP)], idxb, semi).wait()
    pltpu.make_async_copy(idx1_h.at[pl.ds(0, EWP)], idxb2, semi).wait()
    pltpu.make_async_copy(idx1_h.at[pl.ds(0, EWP)], hv, semi).wait()

    _fire_round(ab_h, idxb, va, sema, 0)
    _fire_round(ab_h, idxb2, vb, semb, 0)

    def vbody(i, _):
      off = i * L
      x = va[pl.ds(off, L)] + vb[pl.ds(off, L)]
      e16 = jnp.where(x > 0, x, 0.2 * x)
      ev[pl.ds(off, L)] = e16
      _seg_rmw(hv[pl.ds(off, L)], e16, mloc, kscr, vscr, True)
      return 0

    NVR = G * CH // L

    def rbody(g, _):
      @pl.when(g + 1 < NR)
      def _():
        _fire_round(ab_h, idxb, va, sema, g + 1)
        _fire_round(ab_h, idxb2, vb, semb, g + 1)
      _drain_round(ab_h, va, sema)
      _drain_round(ab_h, vb, semb)
      lax.fori_loop(g * NVR, (g + 1) * NVR, vbody, 0, unroll=False)
      return 0
    lax.fori_loop(0, NR, rbody, 0, unroll=False)

    pltpu.sync_copy(ev, e_o.at[pl.ds(base, EWP)])
    pltpu.sync_copy(mloc, m_o.at[w])

  NPB = NP // CH

  def make_merge(is_max):
    def body(all_ref, out_ref):
      x = all_ref[...]
      out_ref[...] = jnp.max(x, axis=0) if is_max else jnp.sum(x, axis=0)
    return pl.pallas_call(
        body, out_shape=jax.ShapeDtypeStruct((NPB, CH), f32))

  kb_max = make_merge(True)
  kb_sum = make_merge(False)

  @functools.partial(
      pl.kernel,
      out_type=(jax.ShapeDtypeStruct((EP,), f32),
                jax.ShapeDtypeStruct((NW, NP), f32)),
      mesh=mesh,
      compiler_params=pltpu.CompilerParams(needs_layout_passes=False),
      scratch_types=[
          pltpu.VMEM((EWP,), i32),
          pltpu.VMEM((EWP,), f32),
          pltpu.VMEM((EWP,), f32),
          pltpu.VMEM((NP,), f32),
          pltpu.VMEM((L,), i32),
          pltpu.VMEM((L,), f32),
          pltpu.SemaphoreType.DMA,
          pltpu.SemaphoreType.DMA,
      ])
  def kc(e_h, h_h, mg_h, ee_o, s_o, hv, ev, mv, sloc, kscr, vscr, sem, semi):
    w = _wid()
    base = w * EWP
    NR = NCH // G
    pltpu.async_copy(h_h.at[pl.ds(base, EWP)], hv, semi)
    pltpu.async_copy(e_h.at[pl.ds(base, EWP)], ev, semi)

    def zbody(i, _):
      sloc[pl.ds(i * L, L)] = jnp.zeros((L,), f32)
      return 0
    lax.fori_loop(0, NP // L, zbody, 0, unroll=False)

    pltpu.make_async_copy(h_h.at[pl.ds(0, EWP)], hv, semi).wait()
    pltpu.make_async_copy(e_h.at[pl.ds(0, EWP)], ev, semi).wait()

    _fire_round(mg_h, hv, mv, sem, 0)

    def vbody(i, _):
      off = i * L
      ex = jnp.exp(ev[pl.ds(off, L)] - mv[pl.ds(off, L)])
      ev[pl.ds(off, L)] = ex
      _seg_rmw(hv[pl.ds(off, L)], ex, sloc, kscr, vscr, False)
      return 0

    NVR = G * CH // L

    def rbody(g, _):
      @pl.when(g + 1 < NR)
      def _():
        _fire_round(mg_h, hv, mv, sem, g + 1)
      _drain_round(mg_h, mv, sem)
      lax.fori_loop(g * NVR, (g + 1) * NVR, vbody, 0, unroll=False)
      return 0
    lax.fori_loop(0, NR, rbody, 0, unroll=False)

    pltpu.sync_copy(ev, ee_o.at[pl.ds(base, EWP)])
    pltpu.sync_copy(sloc, s_o.at[w])

  CE = 128
  NCHE = EWP // CE
  NB = 2

  @functools.partial(
      pl.kernel,
      out_type=jax.ShapeDtypeStruct((NC, NPO, D), f32),
      mesh=mesh,
      compiler_params=pltpu.CompilerParams(needs_layout_passes=False),
      scratch_types=[
          [pltpu.VMEM((2, CE), i32)] * NB,
          [pltpu.VMEM((CE,), i32)] * NB,
          [pltpu.VMEM((CE,), f32)] * NB,
          [pltpu.VMEM((CE, D), f32)] * NB,
          pltpu.VMEM_SHARED((NPO, D), f32),
          [pltpu.SemaphoreType.DMA] * NB,
          [pltpu.SemaphoreType.DMA] * NB,
          [pltpu.SemaphoreType.DMA] * NB,
      ])
  def ke(ee_h, ht_h, emb_h, zeros_h, out_o,
         htq, hsc, eev, rows, out_sh, sem_i, sem_g, sem_s):
    c = lax.axis_index("c")
    s = lax.axis_index("s")
    w = s * NC + c
    pltpu.sync_copy(zeros_h.at[pl.ds(s * SLC, SLC)],
                    out_sh.at[pl.ds(s * SLC, SLC)])
    plsc.subcore_barrier()

    def fire_idx(j, b):
      pltpu.async_copy(ht_h.at[w, j], htq[b], sem_i[b])
      pltpu.async_copy(ee_h.at[pl.ds(w * EWP + j * CE, CE)], eev[b], sem_i[b])

    def wait_idx(b):
      pltpu.make_async_copy(ht_h.at[0, 0], htq[b], sem_i[b]).wait()
      pltpu.make_async_copy(ee_h.at[pl.ds(0, CE)], eev[b], sem_i[b]).wait()

    def fire_rows(j, b):
      pltpu.async_copy(emb_h.at[htq[b].at[1]], rows[b], sem_g[b])

    def wait_rows(b):
      pltpu.make_async_copy(emb_h.at[pl.ds(0, CE)], rows[b], sem_g[b]).wait()

    def wait_scat(b):
      pltpu.make_async_copy(emb_h.at[pl.ds(0, CE)], rows[b], sem_s[b]).wait()

    fire_idx(0, 0)
    wait_idx(0)
    fire_rows(0, 0)
    fire_idx(1, 1)

    def cbody(g, _):
      for b in range(NB):
        j = g * NB + b
        b1 = (b + 1) % NB

        @pl.when(j + 1 < NCHE)
        def _():
          wait_idx(b1)

        @pl.when(j >= 1)
        def _():
          wait_scat(b1)

        @pl.when(j + 1 < NCHE)
        def _():
          fire_rows(j + 1, b1)

        wait_rows(b)

        ats = []
        for gg in range(CE // L):
          at16 = eev[b][pl.ds(gg * L, L)]
          ats.append(at16)
          hsc[b][pl.ds(gg * L, L)] = htq[b][0, pl.ds(gg * L, L)]

        @pl.when(j + 2 < NCHE)
        def _():
          fire_idx(j + 2, b)

        for gg in range(CE // L):
          at16 = ats[gg]
          for r16 in range(L):
            r = gg * L + r16
            a = at16[r16]
            for kk in range(D // L):
              rows[b][r, pl.ds(kk * L, L)] = rows[b][r, pl.ds(kk * L, L)] * a

        pltpu.async_copy(rows[b], out_sh.at[hsc[b]], sem_s[b], add=True)
      return 0
    lax.fori_loop(0, NCHE // NB, cbody, 0, unroll=False)

    wait_scat((NCHE - 1) % NB)

    plsc.subcore_barrier()
    pltpu.sync_copy(out_sh.at[pl.ds(s * SLC, SLC)],
                    out_o.at[c, pl.ds(s * SLC, SLC)])

  def tc0_body(emb_ref, w_ref, rel_ref, wr_ref, ab_ref):
    dn = (((1,), (1,)), ((), ()))
    wra = lax.dot_general(w_ref[0:D, :], rel_ref[...], dn,
                          preferred_element_type=f32, precision=lax.Precision.HIGHEST)
    wrb = lax.dot_general(w_ref[D:2 * D, :], rel_ref[...], dn,
                          preferred_element_type=f32, precision=lax.Precision.HIGHEST)
    wr = jnp.concatenate([wra, wrb], axis=1)
    wr_ref[...] = wr
    ab_ref[...] = jnp.dot(emb_ref[...], wr, preferred_element_type=f32, precision=lax.Precision.HIGHEST)

  tc0 = pl.pallas_call(
      tc0_body,
      out_shape=(jax.ShapeDtypeStruct((D, 2 * R), f32),
                 jax.ShapeDtypeStruct((N, 2 * R), f32)))

  BN = 1000 if N % 1000 == 0 else N

  def ktc_body(outp_ref, sat_ref, emb_ref, res_ref, wr_ref,
               embn_ref, resn_ref, ab_ref):
    ssum = jnp.sum(sat_ref[...], axis=1, keepdims=True)
    x = (outp_ref[0] + outp_ref[1]) / (ssum + 1e-16) + emb_ref[...]
    n2 = jnp.sum(x * x, axis=-1, keepdims=True)
    y = x / jnp.maximum(jnp.sqrt(n2), 1e-12)
    embn_ref[...] = y
    resn_ref[...] = res_lambda * res_ref[...] + y
    ab_ref[...] = jnp.dot(y, wr_ref[...], preferred_element_type=f32, precision=lax.Precision.HIGHEST)

  ktc = pl.pallas_call(
      ktc_body,
      grid=(N // BN,),
      in_specs=[
          pl.BlockSpec((2, BN, D), lambda i: (0, i, 0)),
          pl.BlockSpec((BN, NW), lambda i: (i, 0)),
          pl.BlockSpec((BN, D), lambda i: (i, 0)),
          pl.BlockSpec((BN, D), lambda i: (i, 0)),
          pl.BlockSpec((D, 2 * R), lambda i: (0, 0)),
      ],
      out_specs=[
          pl.BlockSpec((BN, D), lambda i: (i, 0)),
          pl.BlockSpec((BN, D), lambda i: (i, 0)),
          pl.BlockSpec((BN, 2 * R), lambda i: (i, 0)),
      ],
      out_shape=(jax.ShapeDtypeStruct((N, D), f32),
                 jax.ShapeDtypeStruct((N, D), f32),
                 jax.ShapeDtypeStruct((N, 2 * R), f32)))

  h = edge_index[0]
  t = edge_index[1]
  et = edge_type
  pad = EP - E
  hp = jnp.concatenate([h, jnp.full((pad,), N, i32)])
  tp = jnp.concatenate([t, jnp.zeros((pad,), i32)])
  etp = jnp.concatenate([et, jnp.zeros((pad,), i32)])
  hcl = jnp.minimum(hp, N - 1)
  tcl = jnp.minimum(tp, N - 1)
  idx1 = hcl * (2 * R) + etp
  idx2 = tcl * (2 * R) + R + etp
  ht3 = jnp.stack([hp.reshape(NW, NCHE, CE), tp.reshape(NW, NCHE, CE)],
                  axis=2)
  zeros_np = jnp.zeros((NPO, D), f32)

  wr, ab = tc0(entity_emb, W, relation_emb)
  emb = entity_emb
  res = entity_emb
  for _ in range(n_hops):
    abf = ab.reshape(N * 2 * R)
    e_all, m_all = ka(idx1, idx2, hp, abf)
    m_glob = kb_max(m_all.reshape(NW, NPB, CH)).reshape(NP)
    ee, s_all = kc(e_all, hp, m_glob)
    outp = ke(ee, ht3, emb, zeros_np)
    emb, res, ab = ktc(outp, s_all.T, emb, res, wr)
  return res
